# Initial kernel scaffold; baseline (speedup 1.0000x reference)
import jax
import jax.numpy as jnp
from jax.experimental import pallas as pl


def kernel(x, cond, mask, scores, expert_weights, top_experts, bias):
    raise NotImplementedError("write your pallas kernel here")


# speedup vs baseline: 12.3385x; 12.3385x over previous
import jax
import jax.numpy as jnp
from jax.experimental import pallas as pl

_N_EXPERTS = 8
_TOP_K = 2
_ROWS = 128
_COLS = 128
_BLOCK_T = 256


def _coeff_kernel(te_ref, we_ref, out_ref, *, capacity):
    te = te_ref[:]
    we = we_ref[:]
    row_i = jax.lax.broadcasted_iota(jnp.int32, (_ROWS, _COLS), 0)
    col_i = jax.lax.broadcasted_iota(jnp.int32, (_ROWS, _COLS), 1)
    cum_lane = (row_i <= col_i).astype(jnp.float32)
    pre_row = (col_i < row_i).astype(jnp.float32)
    surv_w = jnp.zeros((_ROWS, _COLS), jnp.float32)
    for e in range(_N_EXPERTS):
        m = (te == e).astype(jnp.float32)
        rowcum = jnp.dot(m, cum_lane, preferred_element_type=jnp.float32)
        row_tot = rowcum[:, _COLS - 1:_COLS]
        prefix = jnp.dot(pre_row, row_tot, preferred_element_type=jnp.float32)
        inc_rank = prefix + rowcum
        keep = (te == e) & (inc_rank <= float(capacity))
        surv_w = surv_w + jnp.where(keep, we, 0.0)
    pc = jax.lax.broadcasted_iota(jnp.int32, (_COLS, _COLS // _TOP_K), 0)
    pj = jax.lax.broadcasted_iota(jnp.int32, (_COLS, _COLS // _TOP_K), 1)
    pair = ((pc // _TOP_K) == pj).astype(jnp.float32)
    out_ref[:] = jnp.dot(surv_w, pair, preferred_element_type=jnp.float32)


def _scale_kernel(x_ref, c_ref, b_ref, o_ref):
    o_ref[:] = x_ref[:] * c_ref[:] + b_ref[:]


def kernel(x, cond, mask, scores, expert_weights, top_experts, bias):
    b, n, d = x.shape
    tk = top_experts.shape[-1]
    T = b * n
    n_slots = T * tk
    capacity = (tk * T) // _N_EXPERTS

    te2d = top_experts.reshape(_ROWS, _COLS)
    we2d = expert_weights.astype(jnp.float32).reshape(_ROWS, _COLS)

    coeff2d = pl.pallas_call(
        lambda te_ref, we_ref, out_ref: _coeff_kernel(
            te_ref, we_ref, out_ref, capacity=capacity),
        out_shape=jax.ShapeDtypeStruct((_ROWS, _COLS // _TOP_K), jnp.float32),
    )(te2d, we2d)

    coeff = coeff2d.reshape(T, 1)
    xf = x.reshape(T, d)
    bias2d = bias.reshape(1, d)

    grid = T // _BLOCK_T
    out = pl.pallas_call(
        _scale_kernel,
        grid=(grid,),
        in_specs=[
            pl.BlockSpec((_BLOCK_T, d), lambda i: (i, 0)),
            pl.BlockSpec((_BLOCK_T, 1), lambda i: (i, 0)),
            pl.BlockSpec((1, d), lambda i: (0, 0)),
        ],
        out_specs=pl.BlockSpec((_BLOCK_T, d), lambda i: (i, 0)),
        out_shape=jax.ShapeDtypeStruct((T, d), jnp.float32),
    )(xf, coeff, bias2d)

    return out.reshape(b, n, d)


# speedup vs baseline: 12.4277x; 1.0072x over previous
import jax
import jax.numpy as jnp
from jax.experimental import pallas as pl

_N_EXPERTS = 8
_TOP_K = 2
_ROWS = 128
_COLS = 128
_BLOCK_T = 512


def _coeff_kernel(te_ref, we_ref, out_ref, *, capacity):
    te = te_ref[:]
    we = we_ref[:]
    row_i = jax.lax.broadcasted_iota(jnp.int32, (_ROWS, _COLS), 0)
    col_i = jax.lax.broadcasted_iota(jnp.int32, (_ROWS, _COLS), 1)
    cum_lane = (row_i <= col_i).astype(jnp.float32)
    pre_row = (col_i < row_i).astype(jnp.float32)
    surv_w = jnp.zeros((_ROWS, _COLS), jnp.float32)
    for e in range(_N_EXPERTS):
        m = (te == e).astype(jnp.float32)
        rowcum = jnp.dot(m, cum_lane, preferred_element_type=jnp.float32)
        row_tot = rowcum[:, _COLS - 1:_COLS]
        prefix = jnp.dot(pre_row, row_tot, preferred_element_type=jnp.float32)
        inc_rank = prefix + rowcum
        keep = (te == e) & (inc_rank <= float(capacity))
        surv_w = surv_w + jnp.where(keep, we, 0.0)
    pc = jax.lax.broadcasted_iota(jnp.int32, (_COLS, _COLS // _TOP_K), 0)
    pj = jax.lax.broadcasted_iota(jnp.int32, (_COLS, _COLS // _TOP_K), 1)
    pair = ((pc // _TOP_K) == pj).astype(jnp.float32)
    out_ref[:] = jnp.dot(surv_w, pair, preferred_element_type=jnp.float32)


def _scale_kernel(x_ref, c_ref, b_ref, o_ref):
    o_ref[:] = x_ref[:] * c_ref[:] + b_ref[:]


def kernel(x, cond, mask, scores, expert_weights, top_experts, bias):
    b, n, d = x.shape
    tk = top_experts.shape[-1]
    T = b * n
    n_slots = T * tk
    capacity = (tk * T) // _N_EXPERTS

    te2d = top_experts.reshape(_ROWS, _COLS)
    we2d = expert_weights.astype(jnp.float32).reshape(_ROWS, _COLS)

    coeff2d = pl.pallas_call(
        lambda te_ref, we_ref, out_ref: _coeff_kernel(
            te_ref, we_ref, out_ref, capacity=capacity),
        out_shape=jax.ShapeDtypeStruct((_ROWS, _COLS // _TOP_K), jnp.float32),
    )(te2d, we2d)

    coeff = coeff2d.reshape(T, 1)
    xf = x.reshape(T, d)
    bias2d = bias.reshape(1, d)

    grid = T // _BLOCK_T
    out = pl.pallas_call(
        _scale_kernel,
        grid=(grid,),
        in_specs=[
            pl.BlockSpec((_BLOCK_T, d), lambda i: (i, 0)),
            pl.BlockSpec((_BLOCK_T, 1), lambda i: (i, 0)),
            pl.BlockSpec((1, d), lambda i: (0, 0)),
        ],
        out_specs=pl.BlockSpec((_BLOCK_T, d), lambda i: (i, 0)),
        out_shape=jax.ShapeDtypeStruct((T, d), jnp.float32),
    )(xf, coeff, bias2d)

    return out.reshape(b, n, d)
